# Initial kernel scaffold; baseline (speedup 1.0000x reference)
import jax
import jax.numpy as jnp
from jax.experimental import pallas as pl


def kernel(points, params):
    raise NotImplementedError("write your pallas kernel here")


# speedup vs baseline: 11.1204x; 11.1204x over previous
import functools

import jax
import jax.numpy as jnp
from jax import lax
from jax.experimental import pallas as pl
from jax.experimental.pallas import tpu as pltpu
from jax.experimental.pallas import tpu_sc as plsc

KNN = 20
NPTS = 2048
NBATCH = 8

_F32 = jnp.float32


def _relu(x):
    return jnp.maximum(x, 0.0)


def _dot(a, b):
    return jnp.dot(a, b, preferred_element_type=_F32)


def _knn_body(xT_ref, x_ref, idx_ref):
    b = pl.program_id(0)
    xT = xT_ref[0]
    x = x_ref[0]
    inner = _dot(xT, x)
    sq_all = jnp.sum(x * x, axis=0, keepdims=True)
    sq_t = jnp.sum(xT * xT, axis=1, keepdims=True)
    dist = sq_t + sq_all - 2.0 * inner
    iota = lax.broadcasted_iota(jnp.int32, dist.shape, 1)
    cur = dist
    cols = []
    for _ in range(KNN):
        m = jnp.min(cur, axis=1, keepdims=True)
        cand = jnp.where(cur == m, iota, NPTS)
        amin = jnp.min(cand, axis=1, keepdims=True)
        cols.append(amin)
        cur = jnp.where(iota == amin, jnp.float32(jnp.inf), cur)
    idx_ref[0] = jnp.concatenate(cols, axis=1) + b * NPTS


def _knn(xT, x, tn=256):
    bsz, n, c = xT.shape
    return pl.pallas_call(
        _knn_body,
        grid=(bsz, n // tn),
        in_specs=[
            pl.BlockSpec((1, tn, c), lambda b, i: (b, i, 0)),
            pl.BlockSpec((1, c, n), lambda b, i: (b, 0, 0)),
        ],
        out_specs=pl.BlockSpec((1, tn, KNN), lambda b, i: (b, i, 0)),
        out_shape=jax.ShapeDtypeStruct((bsz, n, KNN), jnp.int32),
    )(xT, x)


def _gather_rows(table, idxflat):
    m = idxflat.shape[0]
    d = table.shape[1]
    nw = 32
    per_w = m // nw
    chunk = 512
    n_idx_rows = chunk // 128
    n_chunks = per_w // chunk
    idx2d = idxflat.reshape(m // 128, 128)
    mesh = plsc.VectorSubcoreMesh(core_axis_name="c", subcore_axis_name="s")

    @functools.partial(
        pl.kernel,
        mesh=mesh,
        out_type=jax.ShapeDtypeStruct((m, d), _F32),
        scratch_types=[
            pltpu.VMEM((n_idx_rows, 128), jnp.int32),
            pltpu.VMEM((chunk, d), _F32),
            pltpu.SemaphoreType.DMA,
        ],
    )
    def k(table_hbm, idx_hbm, out_hbm, idx_v, rows_v, sem):
        wid = lax.axis_index("s") * 2 + lax.axis_index("c")

        def body(i, carry):
            base = pl.multiple_of(wid * per_w + i * chunk, chunk)
            row0 = pl.multiple_of(base // 128, n_idx_rows)
            pltpu.sync_copy(idx_hbm.at[pl.ds(row0, n_idx_rows)], idx_v)
            descs = []
            for j in range(n_idx_rows):
                descs.append(
                    pltpu.async_copy(
                        table_hbm.at[idx_v.at[j]],
                        rows_v.at[pl.ds(j * 128, 128)],
                        sem,
                    )
                )
            for dsc in descs:
                dsc.wait()
            pltpu.sync_copy(rows_v, out_hbm.at[pl.ds(base, chunk)])
            return carry

        lax.fori_loop(0, n_chunks, body, 0)

    return k(table, idx2d)


def _edge_body(xT_ref, g_ref, w0_ref, b0_ref, w1_ref, b1_ref, o_ref):
    xi = xT_ref[0]
    tn, c = xi.shape
    g = g_ref[0]
    xj = g.reshape(tn, KNN, -1)[:, :, :c]
    d = xj - xi[:, None, :]
    xib = jnp.broadcast_to(xi[:, None, :], (tn, KNN, c))
    e = jnp.concatenate([xib, d], axis=2)
    h = _dot(e.reshape(tn * KNN, 2 * c), w0_ref[...]) + b0_ref[...]
    h = _relu(h)
    h = _dot(h, w1_ref[...]) + b1_ref[...]
    h = _relu(h)
    o_ref[0] = jnp.max(h.reshape(tn, KNN, -1), axis=1)


def _edge(xT, g, w0, b0, w1, b1, tn=256):
    bsz, n, c = xT.shape
    c2 = w1.shape[0]
    return pl.pallas_call(
        _edge_body,
        grid=(bsz, n // tn),
        in_specs=[
            pl.BlockSpec((1, tn, c), lambda b, i: (b, i, 0)),
            pl.BlockSpec((1, tn * KNN, 128), lambda b, i: (b, i, 0)),
            pl.BlockSpec((2 * c, 64), lambda b, i: (0, 0)),
            pl.BlockSpec((1, 64), lambda b, i: (0, 0)),
            pl.BlockSpec((64, c2), lambda b, i: (0, 0)),
            pl.BlockSpec((1, c2), lambda b, i: (0, 0)),
        ],
        out_specs=pl.BlockSpec((1, tn, c2), lambda b, i: (b, i, 0)),
        out_shape=jax.ShapeDtypeStruct((bsz, n, c2), _F32),
    )(xT, g, jnp.transpose(w0), b0.reshape(1, -1),
      jnp.transpose(w1), b1.reshape(1, -1))


def _locmax_body(f_ref, w_ref, b_ref, o_ref):
    nt = pl.program_id(1)
    y = _relu(_dot(f_ref[0], w_ref[...]) + b_ref[...])
    m = jnp.max(y, axis=0, keepdims=True)

    @pl.when(nt == 0)
    def _():
        o_ref[0] = m

    @pl.when(nt > 0)
    def _():
        o_ref[0] = jnp.maximum(o_ref[0], m)


def _locmax(f, w, b, tn=512):
    bsz, n, cin = f.shape
    cout = w.shape[0]
    out = pl.pallas_call(
        _locmax_body,
        grid=(bsz, n // tn),
        in_specs=[
            pl.BlockSpec((1, tn, cin), lambda bb, i: (bb, i, 0)),
            pl.BlockSpec((cin, cout), lambda bb, i: (0, 0)),
            pl.BlockSpec((1, cout), lambda bb, i: (0, 0)),
        ],
        out_specs=pl.BlockSpec((1, 1, cout), lambda bb, i: (bb, 0, 0)),
        out_shape=jax.ShapeDtypeStruct((bsz, 1, cout), _F32),
    )(f, jnp.transpose(w), b.reshape(1, -1))
    return out.reshape(bsz, cout)


def _thead_body(g_ref, pts_ref, w0_ref, b0_ref, w1_ref, b1_ref, wl_ref,
                bl_ref, t9_ref, x0_ref):
    g = g_ref[...]
    y = _relu(_dot(g, w0_ref[...]) + b0_ref[...])
    y = _relu(_dot(y, w1_ref[...]) + b1_ref[...])
    t = _dot(y, wl_ref[...]) + bl_ref[...]
    r = lax.broadcasted_iota(jnp.int32, t.shape, 1)
    eye = jnp.where((r // 3) == (r % 3), 1.0, 0.0).astype(_F32)
    t9_ref[...] = t + eye
    for b in range(NBATCH):
        for i in range(3):
            acc = None
            for j in range(3):
                tb = t9_ref[b, 3 * i + j].astype(jnp.bfloat16).astype(_F32)
                pb = pts_ref[b, j, :].astype(jnp.bfloat16).astype(_F32)
                term = tb * pb
                acc = term if acc is None else acc + term
            x0_ref[b, i, :] = acc


def _tnet_head(gfeat, points, params):
    return pl.pallas_call(
        _thead_body,
        out_shape=[
            jax.ShapeDtypeStruct((NBATCH, 9), _F32),
            jax.ShapeDtypeStruct((NBATCH, 3, NPTS), _F32),
        ],
    )(
        gfeat,
        points,
        jnp.transpose(params["t_g_W0"]), params["t_g_b0"].reshape(1, -1),
        jnp.transpose(params["t_g_W1"]), params["t_g_b1"].reshape(1, -1),
        jnp.transpose(params["t_lin_W"]), params["t_lin_b"].reshape(1, -1),
    )


def _locarg_body(cat_ref, w_ref, b_ref, gf_ref, kpi_ref):
    nt = pl.program_id(1)
    tn = cat_ref.shape[1]
    y = _relu(_dot(cat_ref[0], w_ref[...]) + b_ref[...])
    tm = jnp.max(y, axis=0, keepdims=True)
    rows = lax.broadcasted_iota(jnp.int32, y.shape, 0) + nt * tn
    cand = jnp.where(y == tm, rows, NPTS)
    targ = jnp.min(cand, axis=0, keepdims=True)

    @pl.when(nt == 0)
    def _():
        gf_ref[0] = tm
        kpi_ref[0] = targ

    @pl.when(nt > 0)
    def _():
        better = tm > gf_ref[0]
        kpi_ref[0] = jnp.where(better, targ, kpi_ref[0])
        gf_ref[0] = jnp.maximum(tm, gf_ref[0])


def _loc_argmax(cat, w, b, tn=512):
    bsz, n, cin = cat.shape
    cout = w.shape[0]
    gf, kpi = pl.pallas_call(
        _locarg_body,
        grid=(bsz, n // tn),
        in_specs=[
            pl.BlockSpec((1, tn, cin), lambda bb, i: (bb, i, 0)),
            pl.BlockSpec((cin, cout), lambda bb, i: (0, 0)),
            pl.BlockSpec((1, cout), lambda bb, i: (0, 0)),
        ],
        out_specs=[
            pl.BlockSpec((1, 1, cout), lambda bb, i: (bb, 0, 0)),
            pl.BlockSpec((1, 1, cout), lambda bb, i: (bb, 0, 0)),
        ],
        out_shape=[
            jax.ShapeDtypeStruct((bsz, 1, cout), _F32),
            jax.ShapeDtypeStruct((bsz, 1, cout), jnp.int32),
        ],
    )(cat, jnp.transpose(w), b.reshape(1, -1))
    return gf.reshape(bsz, cout), kpi.reshape(bsz, cout)


def _ghead_body(g_ref, w0_ref, b0_ref, w1_ref, b1_ref, w2_ref, b2_ref,
                wo_ref, bo_ref, o_ref):
    y = _relu(_dot(g_ref[...], w0_ref[...]) + b0_ref[...])
    y = _relu(_dot(y, w1_ref[...]) + b1_ref[...])
    y = _relu(_dot(y, w2_ref[...]) + b2_ref[...])
    o_ref[...] = _dot(y, wo_ref[...]) + bo_ref[...]


def _ghead(gfeat, params):
    return pl.pallas_call(
        _ghead_body,
        out_shape=jax.ShapeDtypeStruct((NBATCH, 16), _F32),
    )(
        gfeat,
        jnp.transpose(params["g_W0"]), params["g_b0"].reshape(1, -1),
        jnp.transpose(params["g_W1"]), params["g_b1"].reshape(1, -1),
        jnp.transpose(params["g_W2"]), params["g_b2"].reshape(1, -1),
        jnp.transpose(params["go_W"]), params["go_b"].reshape(1, -1),
    )


def _seg_body(cat_ref, gf_ref, w0_ref, b0_ref, w1_ref, b1_ref, w2_ref, b2_ref,
              w3_ref, b3_ref, o_ref):
    cat = cat_ref[0]
    tn = cat.shape[0]
    gfb = jnp.broadcast_to(gf_ref[0], (tn, 1024))
    s = jnp.concatenate([gfb, cat], axis=1)
    s = _relu(_dot(s, w0_ref[...]) + b0_ref[...])
    s = _relu(_dot(s, w1_ref[...]) + b1_ref[...])
    s = _relu(_dot(s, w2_ref[...]) + b2_ref[...])
    o_ref[0] = _dot(s, w3_ref[...]) + b3_ref[...]


def _seg(cat, gfeat, params, tn=512):
    bsz, n, cin = cat.shape

    def c(shape):
        return pl.BlockSpec(shape, lambda bb, i: tuple(0 for _ in shape))

    return pl.pallas_call(
        _seg_body,
        grid=(bsz, n // tn),
        in_specs=[
            pl.BlockSpec((1, tn, cin), lambda bb, i: (bb, i, 0)),
            pl.BlockSpec((1, 1, 1024), lambda bb, i: (bb, 0, 0)),
            c((1216, 256)), c((1, 256)),
            c((256, 256)), c((1, 256)),
            c((256, 128)), c((1, 128)),
            c((128, 50)), c((1, 50)),
        ],
        out_specs=pl.BlockSpec((1, tn, 50), lambda bb, i: (bb, i, 0)),
        out_shape=jax.ShapeDtypeStruct((bsz, n, 50), _F32),
    )(
        cat, gfeat.reshape(bsz, 1, 1024),
        jnp.transpose(params["seg_W0"]), params["seg_b0"].reshape(1, -1),
        jnp.transpose(params["seg_W1"]), params["seg_b1"].reshape(1, -1),
        jnp.transpose(params["cs_W"]), params["cs_b"].reshape(1, -1),
        jnp.transpose(params["mo_W"]), params["mo_b"].reshape(1, -1),
    )


def _edge_block(x, xT, w0, b0, w1, b1):
    idx = _knn(xT, x)
    c = xT.shape[2]
    xpad = jnp.pad(xT.reshape(NBATCH * NPTS, c), ((0, 0), (0, 128 - c)))
    g = _gather_rows(xpad, idx.reshape(-1))
    return _edge(xT, g.reshape(NBATCH, NPTS * KNN, 128), w0, b0, w1, b1)


def kernel(points, params):
    pr = params
    ptsT = jnp.transpose(points, (0, 2, 1))

    f = _edge_block(points, ptsT, pr["t_ec_W0"], pr["t_ec_b0"],
                    pr["t_ec_W1"], pr["t_ec_b1"])
    gt = _locmax(f, pr["t_loc_W"], pr["t_loc_b"])
    t9, x0 = _tnet_head(gt, points, pr)
    trans = t9.reshape(NBATCH, 3, 3)

    x = x0
    feats = []
    for i in range(3):
        xT = jnp.transpose(x, (0, 2, 1))
        fo = _edge_block(x, xT, pr["ec%d_W0" % i], pr["ec%d_b0" % i],
                         pr["ec%d_W1" % i], pr["ec%d_b1" % i])
        feats.append(fo)
        x = jnp.transpose(fo, (0, 2, 1))
    cat = jnp.concatenate(feats, axis=2)

    gfeat, key_point_inds = _loc_argmax(cat, pr["loc_W"], pr["loc_b"])
    global_output = _ghead(gfeat, pr)
    segT = _seg(cat, gfeat, pr)
    mask_output = jnp.transpose(segT, (0, 2, 1))
    return global_output, mask_output, trans, key_point_inds


# speedup vs baseline: 14.1160x; 1.2694x over previous
import functools

import jax
import jax.numpy as jnp
from jax import lax
from jax.experimental import pallas as pl
from jax.experimental.pallas import tpu as pltpu
from jax.experimental.pallas import tpu_sc as plsc

KNN = 20
NPTS = 2048
NBATCH = 8

_F32 = jnp.float32


def _relu(x):
    return jnp.maximum(x, 0.0)


def _dot(a, b):
    return jnp.dot(a, b, preferred_element_type=_F32)


def _knn_body(xT_ref, x_ref, idx_ref):
    b = pl.program_id(0)
    xT = xT_ref[0]
    x = x_ref[0]
    inner = _dot(xT, x)
    sq_all = jnp.sum(x * x, axis=0, keepdims=True)
    sq_t = jnp.sum(xT * xT, axis=1, keepdims=True)
    dist = sq_t + sq_all - 2.0 * inner
    iota_f = lax.broadcasted_iota(jnp.int32, dist.shape, 1).astype(_F32)
    cur = dist
    cols = []
    for _ in range(KNN):
        m = jnp.min(cur, axis=1, keepdims=True)
        cand = jnp.where(cur == m, iota_f, jnp.float32(NPTS))
        amin = jnp.min(cand, axis=1, keepdims=True)
        cols.append(amin)
        cur = jnp.where(iota_f == amin, jnp.float32(jnp.inf), cur)
    idxf = jnp.concatenate(cols, axis=1)
    idx_ref[0] = idxf.astype(jnp.int32) + b * NPTS


def _knn(xT, x, tn=256):
    bsz, n, c = xT.shape
    return pl.pallas_call(
        _knn_body,
        grid=(bsz, n // tn),
        in_specs=[
            pl.BlockSpec((1, tn, c), lambda b, i: (b, i, 0)),
            pl.BlockSpec((1, c, n), lambda b, i: (b, 0, 0)),
        ],
        out_specs=pl.BlockSpec((1, tn, KNN), lambda b, i: (b, i, 0)),
        out_shape=jax.ShapeDtypeStruct((bsz, n, KNN), jnp.int32),
    )(xT, x)


def _gather_rows(table, idxflat):
    m = idxflat.shape[0]
    d = table.shape[1]
    nw = 32
    per_w = m // nw
    chunk = 512
    n_idx_rows = chunk // 128
    n_chunks = per_w // chunk
    idx2d = idxflat.reshape(m // 128, 128)
    mesh = plsc.VectorSubcoreMesh(core_axis_name="c", subcore_axis_name="s")

    @functools.partial(
        pl.kernel,
        mesh=mesh,
        out_type=jax.ShapeDtypeStruct((m, d), _F32),
        scratch_types=[
            pltpu.VMEM((n_idx_rows, 128), jnp.int32),
            pltpu.VMEM((chunk, d), _F32),
            pltpu.SemaphoreType.DMA,
        ],
    )
    def k(table_hbm, idx_hbm, out_hbm, idx_v, rows_v, sem):
        wid = lax.axis_index("s") * 2 + lax.axis_index("c")

        def body(i, carry):
            base = pl.multiple_of(wid * per_w + i * chunk, chunk)
            row0 = pl.multiple_of(base // 128, n_idx_rows)
            pltpu.sync_copy(idx_hbm.at[pl.ds(row0, n_idx_rows)], idx_v)
            descs = []
            for j in range(n_idx_rows):
                descs.append(
                    pltpu.async_copy(
                        table_hbm.at[idx_v.at[j]],
                        rows_v.at[pl.ds(j * 128, 128)],
                        sem,
                    )
                )
            for dsc in descs:
                dsc.wait()
            pltpu.sync_copy(rows_v, out_hbm.at[pl.ds(base, chunk)])
            return carry

        lax.fori_loop(0, n_chunks, body, 0)

    return k(table, idx2d)


def _edge_body(xT_ref, g_ref, w0_ref, b0_ref, w1_ref, b1_ref, o_ref):
    xi = xT_ref[0]
    tn, c = xi.shape
    acc = None
    for k in range(KNN):
        xj = g_ref[0, k][:, :c]
        e = jnp.concatenate([xi, xj - xi], axis=1)
        h = _relu(_dot(e, w0_ref[...]) + b0_ref[...])
        h = _relu(_dot(h, w1_ref[...]) + b1_ref[...])
        acc = h if acc is None else jnp.maximum(acc, h)
    o_ref[0] = acc


def _edge(xT, g, w0, b0, w1, b1, tn=256):
    bsz, n, c = xT.shape
    c2 = w1.shape[0]
    return pl.pallas_call(
        _edge_body,
        grid=(bsz, n // tn),
        in_specs=[
            pl.BlockSpec((1, tn, c), lambda b, i: (b, i, 0)),
            pl.BlockSpec((1, KNN, tn, 128), lambda b, i: (b, 0, i, 0)),
            pl.BlockSpec((2 * c, 64), lambda b, i: (0, 0)),
            pl.BlockSpec((1, 64), lambda b, i: (0, 0)),
            pl.BlockSpec((64, c2), lambda b, i: (0, 0)),
            pl.BlockSpec((1, c2), lambda b, i: (0, 0)),
        ],
        out_specs=pl.BlockSpec((1, tn, c2), lambda b, i: (b, i, 0)),
        out_shape=jax.ShapeDtypeStruct((bsz, n, c2), _F32),
    )(xT, g, jnp.transpose(w0), b0.reshape(1, -1),
      jnp.transpose(w1), b1.reshape(1, -1))


def _locmax_body(f_ref, w_ref, b_ref, o_ref):
    nt = pl.program_id(1)
    y = _relu(_dot(f_ref[0], w_ref[...]) + b_ref[...])
    m = jnp.max(y, axis=0, keepdims=True)

    @pl.when(nt == 0)
    def _():
        o_ref[0] = m

    @pl.when(nt > 0)
    def _():
        o_ref[0] = jnp.maximum(o_ref[0], m)


def _locmax(f, w, b, tn=512):
    bsz, n, cin = f.shape
    cout = w.shape[0]
    out = pl.pallas_call(
        _locmax_body,
        grid=(bsz, n // tn),
        in_specs=[
            pl.BlockSpec((1, tn, cin), lambda bb, i: (bb, i, 0)),
            pl.BlockSpec((cin, cout), lambda bb, i: (0, 0)),
            pl.BlockSpec((1, cout), lambda bb, i: (0, 0)),
        ],
        out_specs=pl.BlockSpec((1, 1, cout), lambda bb, i: (bb, 0, 0)),
        out_shape=jax.ShapeDtypeStruct((bsz, 1, cout), _F32),
    )(f, jnp.transpose(w), b.reshape(1, -1))
    return out.reshape(bsz, cout)


def _thead_body(g_ref, pts_ref, w0_ref, b0_ref, w1_ref, b1_ref, wl_ref,
                bl_ref, t9_ref, x0_ref):
    g = g_ref[...]
    y = _relu(_dot(g, w0_ref[...]) + b0_ref[...])
    y = _relu(_dot(y, w1_ref[...]) + b1_ref[...])
    t = _dot(y, wl_ref[...]) + bl_ref[...]
    r = lax.broadcasted_iota(jnp.int32, t.shape, 1)
    eye = jnp.where((r // 3) == (r % 3), 1.0, 0.0).astype(_F32)
    t9_ref[...] = t + eye
    for b in range(NBATCH):
        for i in range(3):
            acc = None
            for j in range(3):
                tb = t9_ref[b, 3 * i + j].astype(jnp.bfloat16).astype(_F32)
                pb = pts_ref[b, j, :].astype(jnp.bfloat16).astype(_F32)
                term = tb * pb
                acc = term if acc is None else acc + term
            x0_ref[b, i, :] = acc


def _tnet_head(gfeat, points, params):
    return pl.pallas_call(
        _thead_body,
        out_shape=[
            jax.ShapeDtypeStruct((NBATCH, 9), _F32),
            jax.ShapeDtypeStruct((NBATCH, 3, NPTS), _F32),
        ],
    )(
        gfeat,
        points,
        jnp.transpose(params["t_g_W0"]), params["t_g_b0"].reshape(1, -1),
        jnp.transpose(params["t_g_W1"]), params["t_g_b1"].reshape(1, -1),
        jnp.transpose(params["t_lin_W"]), params["t_lin_b"].reshape(1, -1),
    )


def _locarg_body(cat_ref, w_ref, b_ref, gf_ref, kpi_ref):
    nt = pl.program_id(1)
    tn = cat_ref.shape[1]
    y = _relu(_dot(cat_ref[0], w_ref[...]) + b_ref[...])
    tm = jnp.max(y, axis=0, keepdims=True)
    rows = lax.broadcasted_iota(jnp.int32, y.shape, 0) + nt * tn
    cand = jnp.where(y == tm, rows, NPTS)
    targ = jnp.min(cand, axis=0, keepdims=True)

    @pl.when(nt == 0)
    def _():
        gf_ref[0] = tm
        kpi_ref[0] = targ

    @pl.when(nt > 0)
    def _():
        better = tm > gf_ref[0]
        kpi_ref[0] = jnp.where(better, targ, kpi_ref[0])
        gf_ref[0] = jnp.maximum(tm, gf_ref[0])


def _loc_argmax(cat, w, b, tn=512):
    bsz, n, cin = cat.shape
    cout = w.shape[0]
    gf, kpi = pl.pallas_call(
        _locarg_body,
        grid=(bsz, n // tn),
        in_specs=[
            pl.BlockSpec((1, tn, cin), lambda bb, i: (bb, i, 0)),
            pl.BlockSpec((cin, cout), lambda bb, i: (0, 0)),
            pl.BlockSpec((1, cout), lambda bb, i: (0, 0)),
        ],
        out_specs=[
            pl.BlockSpec((1, 1, cout), lambda bb, i: (bb, 0, 0)),
            pl.BlockSpec((1, 1, cout), lambda bb, i: (bb, 0, 0)),
        ],
        out_shape=[
            jax.ShapeDtypeStruct((bsz, 1, cout), _F32),
            jax.ShapeDtypeStruct((bsz, 1, cout), jnp.int32),
        ],
    )(cat, jnp.transpose(w), b.reshape(1, -1))
    return gf.reshape(bsz, cout), kpi.reshape(bsz, cout)


def _ghead_body(g_ref, w0_ref, b0_ref, w1_ref, b1_ref, w2_ref, b2_ref,
                wo_ref, bo_ref, o_ref):
    y = _relu(_dot(g_ref[...], w0_ref[...]) + b0_ref[...])
    y = _relu(_dot(y, w1_ref[...]) + b1_ref[...])
    y = _relu(_dot(y, w2_ref[...]) + b2_ref[...])
    o_ref[...] = _dot(y, wo_ref[...]) + bo_ref[...]


def _ghead(gfeat, params):
    return pl.pallas_call(
        _ghead_body,
        out_shape=jax.ShapeDtypeStruct((NBATCH, 16), _F32),
    )(
        gfeat,
        jnp.transpose(params["g_W0"]), params["g_b0"].reshape(1, -1),
        jnp.transpose(params["g_W1"]), params["g_b1"].reshape(1, -1),
        jnp.transpose(params["g_W2"]), params["g_b2"].reshape(1, -1),
        jnp.transpose(params["go_W"]), params["go_b"].reshape(1, -1),
    )


def _seg_body(cat_ref, gf_ref, w0_ref, b0_ref, w1_ref, b1_ref, w2_ref, b2_ref,
              w3_ref, b3_ref, o_ref):
    cat = cat_ref[0]
    tn = cat.shape[0]
    gfb = jnp.broadcast_to(gf_ref[0], (tn, 1024))
    s = jnp.concatenate([gfb, cat], axis=1)
    s = _relu(_dot(s, w0_ref[...]) + b0_ref[...])
    s = _relu(_dot(s, w1_ref[...]) + b1_ref[...])
    s = _relu(_dot(s, w2_ref[...]) + b2_ref[...])
    o_ref[0] = _dot(s, w3_ref[...]) + b3_ref[...]


def _seg(cat, gfeat, params, tn=512):
    bsz, n, cin = cat.shape

    def c(shape):
        return pl.BlockSpec(shape, lambda bb, i: tuple(0 for _ in shape))

    return pl.pallas_call(
        _seg_body,
        grid=(bsz, n // tn),
        in_specs=[
            pl.BlockSpec((1, tn, cin), lambda bb, i: (bb, i, 0)),
            pl.BlockSpec((1, 1, 1024), lambda bb, i: (bb, 0, 0)),
            c((1216, 256)), c((1, 256)),
            c((256, 256)), c((1, 256)),
            c((256, 128)), c((1, 128)),
            c((128, 50)), c((1, 50)),
        ],
        out_specs=pl.BlockSpec((1, tn, 50), lambda bb, i: (bb, i, 0)),
        out_shape=jax.ShapeDtypeStruct((bsz, n, 50), _F32),
    )(
        cat, gfeat.reshape(bsz, 1, 1024),
        jnp.transpose(params["seg_W0"]), params["seg_b0"].reshape(1, -1),
        jnp.transpose(params["seg_W1"]), params["seg_b1"].reshape(1, -1),
        jnp.transpose(params["cs_W"]), params["cs_b"].reshape(1, -1),
        jnp.transpose(params["mo_W"]), params["mo_b"].reshape(1, -1),
    )


def _edge_block(x, xT, w0, b0, w1, b1):
    idx = _knn(xT, x)
    c = xT.shape[2]
    xpad = jnp.pad(xT.reshape(NBATCH * NPTS, c), ((0, 0), (0, 128 - c)))
    idx_km = jnp.transpose(idx, (0, 2, 1)).reshape(-1)
    g = _gather_rows(xpad, idx_km)
    return _edge(xT, g.reshape(NBATCH, KNN, NPTS, 128), w0, b0, w1, b1)


def kernel(points, params):
    pr = params
    ptsT = jnp.transpose(points, (0, 2, 1))

    f = _edge_block(points, ptsT, pr["t_ec_W0"], pr["t_ec_b0"],
                    pr["t_ec_W1"], pr["t_ec_b1"])
    gt = _locmax(f, pr["t_loc_W"], pr["t_loc_b"])
    t9, x0 = _tnet_head(gt, points, pr)
    trans = t9.reshape(NBATCH, 3, 3)

    x = x0
    feats = []
    for i in range(3):
        xT = jnp.transpose(x, (0, 2, 1))
        fo = _edge_block(x, xT, pr["ec%d_W0" % i], pr["ec%d_b0" % i],
                         pr["ec%d_W1" % i], pr["ec%d_b1" % i])
        feats.append(fo)
        x = jnp.transpose(fo, (0, 2, 1))
    cat = jnp.concatenate(feats, axis=2)

    gfeat, key_point_inds = _loc_argmax(cat, pr["loc_W"], pr["loc_b"])
    global_output = _ghead(gfeat, pr)
    segT = _seg(cat, gfeat, pr)
    mask_output = jnp.transpose(segT, (0, 2, 1))
    return global_output, mask_output, trans, key_point_inds
